# baseline (speedup 1.0000x reference)
import jax
import jax.numpy as jnp
from jax.experimental import pallas as pl

_EPS = 1e-5
_ROWS = 4096


def _body(a_ref, w_ref, bias_ref, g_ref, beta_ref, o_ref):
    w = w_ref[...]
    bb = bias_ref[...]
    g = g_ref[...]
    beta = beta_ref[...]
    wbar = jnp.mean(w)
    bbar = jnp.mean(bb)
    dw = w - wbar
    db = bb - bbar
    p = jnp.mean(dw * dw)
    q = jnp.mean(dw * db)
    r = jnp.mean(db * db)
    va = dw * g
    vb = db * g
    a = a_ref[...]
    s = jax.lax.rsqrt((a * a) * p + a * (2.0 * q) + (r + _EPS))
    o_ref[...] = (a * s) * va + s * vb + beta


def kernel(data, time, weekday, W, b, ln_gamma, ln_beta):
    del time, weekday
    bsz, seq, nodes, _ = data.shape
    size = W.shape[0]
    m = bsz * seq * nodes
    a2 = data.reshape(m, 1)
    row = lambda v: v.reshape(1, size)
    vec_spec = pl.BlockSpec((1, size), lambda i: (0, 0))
    out = pl.pallas_call(
        _body,
        grid=(m // _ROWS,),
        in_specs=[
            pl.BlockSpec((_ROWS, 1), lambda i: (i, 0)),
            vec_spec, vec_spec, vec_spec, vec_spec,
        ],
        out_specs=pl.BlockSpec((_ROWS, size), lambda i: (i, 0)),
        out_shape=jax.ShapeDtypeStruct((m, size), jnp.float32),
    )(a2, row(W), row(b), row(ln_gamma), row(ln_beta))
    return out.reshape(bsz, seq, nodes, size)
